# Initial kernel scaffold; baseline (speedup 1.0000x reference)
import jax
import jax.numpy as jnp
from jax.experimental import pallas as pl


def kernel(node_features, pos, edge_index, edge_attr, W_msg_x, W_msg_e, W_msg_out, W_upd_x, W_upd_m, W_upd_out):
    raise NotImplementedError("write your pallas kernel here")


# speedup vs baseline: 3.1717x; 3.1717x over previous
import functools

import jax
import jax.numpy as jnp
from jax import lax
from jax.experimental import pallas as pl
from jax.experimental.pallas import tpu as pltpu
from jax.experimental.pallas import tpu_sc as plsc

N = 10000
E = 320000
D = 128
DE = 16
DH = 256
DO = 128

NC = 2
NS = 16
NW = NC * NS
EPW = E // NW
CH = 80
NT = EPW // CH


def _sc_mesh():
    return plsc.VectorSubcoreMesh(
        core_axis_name="c", subcore_axis_name="s", num_cores=NC, num_subcores=NS
    )


def _sc_gather(nf, row2d, col2d):
    @functools.partial(
        pl.kernel,
        out_type=[
            jax.ShapeDtypeStruct((E, D), jnp.float32),
            jax.ShapeDtypeStruct((E, D), jnp.float32),
        ],
        mesh=_sc_mesh(),
        scratch_types=[
            pltpu.VMEM((NT, CH), jnp.int32),
            pltpu.VMEM((NT, CH), jnp.int32),
            pltpu.VMEM((CH, D), jnp.float32),
            pltpu.VMEM((CH, D), jnp.float32),
            pltpu.SemaphoreType.DMA,
            pltpu.SemaphoreType.DMA,
        ],
    )
    def k(nf_hbm, row_hbm, col_hbm, xr_hbm, xc_hbm,
          ridx_v, cidx_v, xr_v, xc_v, sem0, sem1):
        wid = lax.axis_index("c") * NS + lax.axis_index("s")
        pltpu.sync_copy(row_hbm.at[wid], ridx_v)
        pltpu.sync_copy(col_hbm.at[wid], cidx_v)

        def body(t, carry):
            off = pl.multiple_of(wid * EPW + t * CH, 8)
            cp0 = pltpu.async_copy(nf_hbm.at[ridx_v.at[t]], xr_v, sem0)
            cp1 = pltpu.async_copy(nf_hbm.at[cidx_v.at[t]], xc_v, sem1)
            cp0.wait()
            cp1.wait()
            pltpu.sync_copy(xr_v, xr_hbm.at[pl.ds(off, CH)])
            pltpu.sync_copy(xc_v, xc_hbm.at[pl.ds(off, CH)])
            return carry

        lax.fori_loop(0, NT, body, 0)

    return k(nf, row2d, col2d)


def _sc_scatter(messages, col2d, zeros):
    @functools.partial(
        pl.kernel,
        out_type=jax.ShapeDtypeStruct((NC * N, D), jnp.float32),
        mesh=_sc_mesh(),
        scratch_types=[
            pltpu.VMEM((NT, CH), jnp.int32),
            pltpu.VMEM((CH, D), jnp.float32),
            pltpu.VMEM_SHARED((N, D), jnp.float32),
        ],
    )
    def k(msg_hbm, col_hbm, zero_hbm, out_hbm, cidx_v, msg_v, agg_sh):
        cid = lax.axis_index("c")
        sid = lax.axis_index("s")
        wid = cid * NS + sid
        rz = 632
        zoff = pl.multiple_of(jnp.where(sid == NS - 1, N - rz, sid * rz), 8)
        pltpu.sync_copy(zero_hbm.at[pl.ds(zoff, rz)],
                        agg_sh.at[pl.ds(zoff, rz)])
        plsc.subcore_barrier()

        pltpu.sync_copy(col_hbm.at[wid], cidx_v)

        def body(t, carry):
            off = pl.multiple_of(wid * EPW + t * CH, 8)
            pltpu.sync_copy(msg_hbm.at[pl.ds(off, CH)], msg_v)
            pltpu.sync_copy(msg_v, agg_sh.at[cidx_v.at[t]], add=True)
            return carry

        lax.fori_loop(0, NT, body, 0)
        plsc.subcore_barrier()
        pltpu.sync_copy(agg_sh.at[pl.ds(zoff, rz)],
                        out_hbm.at[pl.ds(pl.multiple_of(cid * N + zoff, 8), rz)])

    return k(messages, col2d, zeros)


def _tc_messages(xr, xc, ea, wx, we, wo):
    BE = 2000

    def body(xr_ref, xc_ref, ea_ref, wx_ref, we_ref, wo_ref, out_ref):
        p = xr_ref[...] * xc_ref[...]
        z = jnp.dot(p, wx_ref[...], preferred_element_type=jnp.float32)
        g = jnp.dot(ea_ref[...], we_ref[...], preferred_element_type=jnp.float32)
        z = z * g
        z = z * (1.0 / (1.0 + jnp.exp(-z)))
        out_ref[...] = jnp.dot(z, wo_ref[...], preferred_element_type=jnp.float32)

    return pl.pallas_call(
        body,
        grid=(E // BE,),
        in_specs=[
            pl.BlockSpec((BE, D), lambda i: (i, 0)),
            pl.BlockSpec((BE, D), lambda i: (i, 0)),
            pl.BlockSpec((BE, DE), lambda i: (i, 0)),
            pl.BlockSpec((D, DH), lambda i: (0, 0)),
            pl.BlockSpec((DE, DH), lambda i: (0, 0)),
            pl.BlockSpec((DH, DO), lambda i: (0, 0)),
        ],
        out_specs=pl.BlockSpec((BE, DO), lambda i: (i, 0)),
        out_shape=jax.ShapeDtypeStruct((E, DO), jnp.float32),
    )(xr, xc, ea, wx, we, wo)


def _tc_update(x, agg2, wx, wm, wo):
    BN = 1000
    nblk = N // BN

    def body(x_ref, a0_ref, a1_ref, wx_ref, wm_ref, wo_ref, out_ref):
        a = a0_ref[...] + a1_ref[...]
        u = jnp.dot(x_ref[...], wx_ref[...], preferred_element_type=jnp.float32)
        u = u * jnp.dot(a, wm_ref[...], preferred_element_type=jnp.float32)
        u = u * (1.0 / (1.0 + jnp.exp(-u)))
        out_ref[...] = jnp.dot(u, wo_ref[...], preferred_element_type=jnp.float32)

    return pl.pallas_call(
        body,
        grid=(nblk,),
        in_specs=[
            pl.BlockSpec((BN, D), lambda i: (i, 0)),
            pl.BlockSpec((BN, D), lambda i: (i, 0)),
            pl.BlockSpec((BN, D), lambda i: (i + nblk, 0)),
            pl.BlockSpec((D, DH), lambda i: (0, 0)),
            pl.BlockSpec((DO, DH), lambda i: (0, 0)),
            pl.BlockSpec((DH, DO), lambda i: (0, 0)),
        ],
        out_specs=pl.BlockSpec((BN, DO), lambda i: (i, 0)),
        out_shape=jax.ShapeDtypeStruct((N, DO), jnp.float32),
    )(x, agg2, agg2, wx, wm, wo)


def kernel(node_features, pos, edge_index, edge_attr,
           W_msg_x, W_msg_e, W_msg_out, W_upd_x, W_upd_m, W_upd_out):
    del pos
    row2d = edge_index[0].astype(jnp.int32).reshape(NW, NT, CH)
    col2d = edge_index[1].astype(jnp.int32).reshape(NW, NT, CH)
    xr, xc = _sc_gather(node_features, row2d, col2d)
    messages = _tc_messages(xr, xc, edge_attr, W_msg_x, W_msg_e, W_msg_out)
    zeros = jnp.zeros((N, D), jnp.float32)
    agg2 = _sc_scatter(messages, col2d, zeros)
    return _tc_update(node_features, agg2, W_upd_x, W_upd_m, W_upd_out)
